# baseline (speedup 1.0000x reference)
import jax
import jax.numpy as jnp
from jax.experimental import pallas as pl
from jax.experimental.pallas import tpu as pltpu

_BK = 512


def _router_block(x_ref, w_ref, noise_ref, o_ref):
    k = pl.program_id(0)
    psum = jax.lax.dot_general(
        x_ref[...].astype(jnp.bfloat16),
        w_ref[...].astype(jnp.bfloat16),
        dimension_numbers=(((1,), (1,)), ((), ())),
        preferred_element_type=jnp.float32,
    )

    @pl.when(k == 0)
    def _init():
        o_ref[...] = psum + noise_ref[...]

    @pl.when(k > 0)
    def _acc():
        o_ref[...] += psum


@jax.jit
def kernel(x, W, noise):
    tokens, d_model = x.shape
    n_experts = W.shape[0]
    grid = (d_model // _BK,)
    return pl.pallas_call(
        _router_block,
        grid=grid,
        in_specs=[
            pl.BlockSpec((tokens, _BK), lambda k: (0, k)),
            pl.BlockSpec((n_experts, _BK), lambda k: (0, k)),
            pl.BlockSpec((tokens, n_experts), lambda k: (0, 0)),
        ],
        out_specs=pl.BlockSpec((tokens, n_experts), lambda k: (0, 0)),
        out_shape=jax.ShapeDtypeStruct((tokens, n_experts), jnp.float32),
        compiler_params=pltpu.CompilerParams(
            dimension_semantics=("arbitrary",),
        ),
    )(x, W, noise)
